# baseline (speedup 1.0000x reference)
import jax
import jax.numpy as jnp
from jax import lax
from jax.experimental import pallas as pl
from jax.experimental.pallas import tpu as pltpu
from jax.experimental.pallas import tpu_sc as plsc

_NC = 2
_NS = 16
_NW = _NC * _NS
_L = 16

_N_CELLS = 16384
_D = 64
_BPW = _N_CELLS // _NW
_IDX_CHUNK = 128
_NCHUNK = _BPW // _IDX_CHUNK


def _body(z_hbm, idx_hbm, table_hbm, out_hbm, idx_v, acc_v, rows_v, sem):
    wid = lax.axis_index("s") * _NC + lax.axis_index("c")
    base = wid * _BPW

    pltpu.sync_copy(idx_hbm.at[pl.ds(base, _BPW)], idx_v)

    copies = [
        pltpu.async_copy(
            table_hbm.at[idx_v.at[pl.ds(j * _IDX_CHUNK, _IDX_CHUNK)]],
            rows_v.at[pl.ds(j * _IDX_CHUNK, _IDX_CHUNK)],
            sem,
        )
        for j in range(_NCHUNK)
    ]

    pltpu.sync_copy(z_hbm.at[pl.ds(base, _BPW)], acc_v)

    for cp in copies:
        cp.wait()

    def row_add(r, carry):
        for c in range(_D // _L):
            sl = pl.ds(c * _L, _L)
            plsc.addupdate(acc_v.at[r, sl], rows_v[r, sl])
        return carry

    lax.fori_loop(0, _BPW, row_add, 0, unroll=8)

    pltpu.sync_copy(acc_v, out_hbm.at[pl.ds(base, _BPW)])


@jax.jit
def kernel(z_bio, batch_ids, emb_weight):
    idx = batch_ids if batch_ids.dtype == jnp.int32 else batch_ids.astype(jnp.int32)
    mesh = plsc.VectorSubcoreMesh(
        core_axis_name="c", subcore_axis_name="s",
        num_cores=_NC, num_subcores=_NS,
    )
    f = pl.kernel(
        _body,
        out_type=jax.ShapeDtypeStruct((_N_CELLS, _D), jnp.float32),
        mesh=mesh,
        scratch_types=[
            pltpu.VMEM((_BPW,), jnp.int32),
            pltpu.VMEM((_BPW, _D), jnp.float32),
            pltpu.VMEM((_BPW, _D), jnp.float32),
            pltpu.SemaphoreType.DMA,
        ],
        compiler_params=pltpu.CompilerParams(
            use_tc_tiling_on_sc=False,
            skip_device_barrier=True,
        ),
    )
    return f(z_bio, idx, emb_weight)


# speedup vs baseline: 2.2609x; 2.2609x over previous
import jax
import jax.numpy as jnp
from jax import lax
from jax.experimental import pallas as pl
from jax.experimental.pallas import tpu as pltpu
from jax.experimental.pallas import tpu_sc as plsc

_NC = 2
_NS = 16
_NW = _NC * _NS
_L = 16

_N_CELLS = 16384
_D = 64
_VOCAB = 100000
_RPW = _D // _NW
_ICHUNK = 2048
_NICHUNK = _N_CELLS // _ICHUNK


def _body(z_hbm, idx_hbm, table_hbm, out_hbm, row_v, acc_v, idx_a, idx_b, sems):
    wid = lax.axis_index("s") * _NC + lax.axis_index("c")

    for r in range(_RPW):
        j = wid * _RPW + r
        rowcp = pltpu.async_copy(table_hbm.at[pl.ds(j, 1), :], row_v, sems[2])
        zcp = pltpu.async_copy(z_hbm.at[pl.ds(j, 1), :], acc_v, sems[3])
        icp = pltpu.async_copy(idx_hbm.at[pl.ds(0, _ICHUNK)], idx_a, sems[0])
        rowcp.wait()
        zcp.wait()

        for ch in range(_NICHUNK):
            if ch + 1 < _NICHUNK:
                nxt = pltpu.async_copy(
                    idx_hbm.at[pl.ds((ch + 1) * _ICHUNK, _ICHUNK)],
                    (idx_a, idx_b)[(ch + 1) % 2], sems[(ch + 1) % 2])
            icp.wait()
            buf = (idx_a, idx_b)[ch % 2]
            base = ch * _ICHUNK

            def grp(g, carry, buf=buf, base=base):
                k = g * _L
                iv = buf[pl.ds(k, _L)]
                gathered = plsc.load_gather(row_v, [jnp.zeros((_L,), jnp.int32), iv])
                plsc.addupdate(acc_v.at[0, pl.ds(base + k, _L)], gathered)
                return carry

            lax.fori_loop(0, _ICHUNK // _L, grp, 0, unroll=8)
            if ch + 1 < _NICHUNK:
                icp = nxt

        pltpu.sync_copy(acc_v, out_hbm.at[pl.ds(j, 1), :])


@jax.jit
def kernel(z_bio, batch_ids, emb_weight):
    idx = batch_ids if batch_ids.dtype == jnp.int32 else batch_ids.astype(jnp.int32)
    zt = z_bio.T
    tt = emb_weight.T
    mesh = plsc.VectorSubcoreMesh(
        core_axis_name="c", subcore_axis_name="s",
        num_cores=_NC, num_subcores=_NS,
    )
    f = pl.kernel(
        _body,
        out_type=jax.ShapeDtypeStruct((_D, _N_CELLS), jnp.float32),
        mesh=mesh,
        scratch_types=[
            pltpu.VMEM((1, _VOCAB), jnp.float32),
            pltpu.VMEM((1, _N_CELLS), jnp.float32),
            pltpu.VMEM((_ICHUNK,), jnp.int32),
            pltpu.VMEM((_ICHUNK,), jnp.int32),
            [pltpu.SemaphoreType.DMA] * 4,
        ],
        compiler_params=pltpu.CompilerParams(
            use_tc_tiling_on_sc=True,
            skip_device_barrier=True,
            needs_layout_passes=False,
        ),
    )
    return f(zt, idx, tt).T
